# Initial kernel scaffold; baseline (speedup 1.0000x reference)
import jax
import jax.numpy as jnp
from jax.experimental import pallas as pl


def kernel(xyz, features, sampled_xyz, knn_indices, valid_knn_mask, w1, b1, w2, b2, w_lin, b_lin):
    raise NotImplementedError("write your pallas kernel here")


# speedup vs baseline: 4.9137x; 4.9137x over previous
import functools

import jax
import jax.numpy as jnp
from jax import lax
from jax.experimental import pallas as pl
from jax.experimental.pallas import tpu as pltpu
from jax.experimental.pallas import tpu_sc as plsc

_NW = 32
_RPD = 128


def _make_gather(nb, nd):
    per_w = nb // _NW
    ndma = per_w // _RPD

    @functools.partial(
        pl.kernel,
        mesh=plsc.VectorSubcoreMesh(core_axis_name="c", subcore_axis_name="s"),
        out_type=jax.ShapeDtypeStruct((nb, nd), jnp.float32),
        scratch_types=[
            pltpu.VMEM((ndma, _RPD), jnp.int32),
            pltpu.VMEM((_RPD, nd), jnp.float32),
            pltpu.SemaphoreType.DMA,
        ],
        compiler_params=pltpu.CompilerParams(use_tc_tiling_on_sc=False),
    )
    def gather_kernel(tbl_hbm, idx_hbm, out_hbm, idx_v, rows_v, sem):
        wid = lax.axis_index("s") * 2 + lax.axis_index("c")
        pltpu.sync_copy(idx_hbm.at[pl.ds(wid * ndma, ndma)], idx_v)
        base = wid * per_w

        def body(j, carry):
            pltpu.async_copy(tbl_hbm.at[idx_v.at[j]], rows_v, sem).wait()
            pltpu.sync_copy(rows_v, out_hbm.at[pl.ds(base + j * _RPD, _RPD)])
            return carry

        lax.fori_loop(0, ndma, body, 0)

    return gather_kernel


def _tc_body(g_ref, samp_ref, w1_ref, b1_ref, w2_ref, b2_ref, wl_ref, bl_ref,
             out_ref, *, mt, kk, nd, nh, nj):
    g = g_ref[...]
    g3 = g.reshape(mt, kk, nd)
    s = samp_ref[...]
    xyzn = g3[:, :, 0:3] - s[:, None, :]
    x2 = xyzn.reshape(mt * kk, 3)
    hid = jnp.dot(x2, w1_ref[...], preferred_element_type=jnp.float32)
    hid = hid + b1_ref[...]
    hid = jnp.where(hid >= 0, hid, 0.1 * hid)
    wts = jnp.dot(hid, w2_ref[...], preferred_element_type=jnp.float32)
    wts = wts + b2_ref[...]
    wts = jnp.where(wts >= 0, wts, 0.1 * wts)
    w3 = wts.reshape(mt, kk, nj)
    acc = w3[:, 0, :, None] * g3[:, 0, None, :]
    for k in range(1, kk):
        acc = acc + w3[:, k, :, None] * g3[:, k, None, :]
    flat = acc.reshape(mt, nj * nd)
    o = jnp.dot(flat, wl_ref[...], preferred_element_type=jnp.float32)
    o = o + bl_ref[...]
    out_ref[...] = jnp.where(o >= 0, o, 0.1 * o)


def kernel(xyz, features, sampled_xyz, knn_indices, valid_knn_mask,
           w1, b1, w2, b2, w_lin, b_lin):
    B, C, H, W = features.shape
    hh, ww = sampled_xyz.shape[2], sampled_xyz.shape[3]
    N = H * W
    M = hh * ww
    K = knn_indices.shape[2]
    CF = C + 3
    ND = 48
    NH = w1.shape[0]
    NJ = w2.shape[0]
    OC = w_lin.shape[0]

    feats = jnp.concatenate([xyz.reshape(B, 3, N), features.reshape(B, C, N)], axis=1)
    tbl = jnp.transpose(feats, (0, 2, 1)).reshape(B * N, CF)
    tbl = jnp.pad(tbl, ((0, 8), (0, ND - CF)))

    offs = (jnp.arange(B, dtype=jnp.int32) * N)[:, None, None]
    idx = knn_indices.astype(jnp.int32) + offs
    idx = jnp.where(valid_knn_mask, idx, B * N)
    idx2 = idx.reshape(-1, _RPD)

    g = _make_gather(B * M * K, ND)(tbl, idx2)

    samp = jnp.transpose(sampled_xyz.reshape(B, 3, M), (0, 2, 1)).reshape(B * M, 3)
    w1t = w1.T
    w2t = w2.T
    wlp = jnp.pad(w_lin.reshape(OC, NJ, CF), ((0, 0), (0, 0), (0, ND - CF)))
    wlp = wlp.reshape(OC, NJ * ND).T
    b1r = b1.reshape(1, NH)
    b2r = b2.reshape(1, NJ)
    blr = b_lin.reshape(1, OC)

    MT = 256
    grid = (B * M // MT,)
    o = pl.pallas_call(
        functools.partial(_tc_body, mt=MT, kk=K, nd=ND, nh=NH, nj=NJ),
        grid=grid,
        in_specs=[
            pl.BlockSpec((MT * K, ND), lambda i: (i, 0)),
            pl.BlockSpec((MT, 3), lambda i: (i, 0)),
            pl.BlockSpec((3, NH), lambda i: (0, 0)),
            pl.BlockSpec((1, NH), lambda i: (0, 0)),
            pl.BlockSpec((NH, NJ), lambda i: (0, 0)),
            pl.BlockSpec((1, NJ), lambda i: (0, 0)),
            pl.BlockSpec((NJ * ND, OC), lambda i: (0, 0)),
            pl.BlockSpec((1, OC), lambda i: (0, 0)),
        ],
        out_specs=pl.BlockSpec((MT, OC), lambda i: (i, 0)),
        out_shape=jax.ShapeDtypeStruct((B * M, OC), jnp.float32),
    )(g, samp, w1t, b1r, w2t, b2r, wlp, blr)

    return jnp.transpose(o.reshape(B, M, OC), (0, 2, 1)).reshape(B, OC, hh, ww)


# speedup vs baseline: 9.1190x; 1.8558x over previous
import functools

import jax
import jax.numpy as jnp
from jax import lax
from jax.experimental import pallas as pl
from jax.experimental.pallas import tpu as pltpu
from jax.experimental.pallas import tpu_sc as plsc

_NW = 32
_RPD = 128


def _make_gather(nb, nd):
    per_w = nb // _NW
    ndma = per_w // _RPD

    @functools.partial(
        pl.kernel,
        mesh=plsc.VectorSubcoreMesh(core_axis_name="c", subcore_axis_name="s"),
        out_type=jax.ShapeDtypeStruct((nb, nd), jnp.float32),
        scratch_types=[
            pltpu.VMEM((ndma, _RPD), jnp.int32),
            pltpu.VMEM((_RPD, nd), jnp.float32),
            pltpu.SemaphoreType.DMA,
        ],
        compiler_params=pltpu.CompilerParams(use_tc_tiling_on_sc=False),
    )
    def gather_kernel(tbl_hbm, idx_hbm, out_hbm, idx_v, rows_v, sem):
        wid = lax.axis_index("s") * 2 + lax.axis_index("c")
        pltpu.sync_copy(idx_hbm.at[pl.ds(wid * ndma, ndma)], idx_v)
        base = wid * per_w

        def body(j, carry):
            pltpu.async_copy(tbl_hbm.at[idx_v.at[j]], rows_v, sem).wait()
            pltpu.sync_copy(rows_v, out_hbm.at[pl.ds(base + j * _RPD, _RPD)])
            return carry

        lax.fori_loop(0, ndma, body, 0)

    return gather_kernel


def _tc_body(g_ref, samp_ref, w1_ref, b1_ref, w2_ref, b2_ref, wl_ref, bl_ref,
             out_ref, *, mt, kk, nd, nh, nj):
    g = g_ref[...]
    g3 = g.reshape(mt, kk, nd)
    s = samp_ref[...]
    xyzn = g3[:, :, 0:3] - s[:, None, :]
    x2 = xyzn.reshape(mt * kk, 3)
    hid = jnp.dot(x2, w1_ref[...], preferred_element_type=jnp.float32)
    hid = hid + b1_ref[...]
    hid = jnp.where(hid >= 0, hid, 0.1 * hid)
    wts = jnp.dot(hid, w2_ref[...], preferred_element_type=jnp.float32)
    wts = wts + b2_ref[...]
    wts = jnp.where(wts >= 0, wts, 0.1 * wts)
    w3 = wts.reshape(mt, kk, nj)
    acc = jax.lax.dot_general(
        w3, g3, (((1,), (1,)), ((0,), (0,))),
        preferred_element_type=jnp.float32)
    flat = acc.reshape(mt, nj * nd)
    o = jnp.dot(flat, wl_ref[...], preferred_element_type=jnp.float32)
    o = o + bl_ref[...]
    out_ref[...] = jnp.where(o >= 0, o, 0.1 * o)


def kernel(xyz, features, sampled_xyz, knn_indices, valid_knn_mask,
           w1, b1, w2, b2, w_lin, b_lin):
    B, C, H, W = features.shape
    hh, ww = sampled_xyz.shape[2], sampled_xyz.shape[3]
    N = H * W
    M = hh * ww
    K = knn_indices.shape[2]
    CF = C + 3
    ND = 48
    NH = w1.shape[0]
    NJ = w2.shape[0]
    OC = w_lin.shape[0]

    feats = jnp.concatenate([xyz.reshape(B, 3, N), features.reshape(B, C, N)], axis=1)
    tbl = jnp.transpose(feats, (0, 2, 1)).reshape(B * N, CF)
    tbl = jnp.pad(tbl, ((0, 8), (0, ND - CF)))

    offs = (jnp.arange(B, dtype=jnp.int32) * N)[:, None, None]
    idx = knn_indices.astype(jnp.int32) + offs
    idx = jnp.where(valid_knn_mask, idx, B * N)
    idx2 = idx.reshape(-1, _RPD)

    g = _make_gather(B * M * K, ND)(tbl, idx2)

    samp = jnp.transpose(sampled_xyz.reshape(B, 3, M), (0, 2, 1)).reshape(B * M, 3)
    w1t = w1.T
    w2t = w2.T
    wlp = jnp.pad(w_lin.reshape(OC, NJ, CF), ((0, 0), (0, 0), (0, ND - CF)))
    wlp = wlp.reshape(OC, NJ * ND).T
    b1r = b1.reshape(1, NH)
    b2r = b2.reshape(1, NJ)
    blr = b_lin.reshape(1, OC)

    MT = 256
    grid = (B * M // MT,)
    o = pl.pallas_call(
        functools.partial(_tc_body, mt=MT, kk=K, nd=ND, nh=NH, nj=NJ),
        grid=grid,
        in_specs=[
            pl.BlockSpec((MT * K, ND), lambda i: (i, 0)),
            pl.BlockSpec((MT, 3), lambda i: (i, 0)),
            pl.BlockSpec((3, NH), lambda i: (0, 0)),
            pl.BlockSpec((1, NH), lambda i: (0, 0)),
            pl.BlockSpec((NH, NJ), lambda i: (0, 0)),
            pl.BlockSpec((1, NJ), lambda i: (0, 0)),
            pl.BlockSpec((NJ * ND, OC), lambda i: (0, 0)),
            pl.BlockSpec((1, OC), lambda i: (0, 0)),
        ],
        out_specs=pl.BlockSpec((MT, OC), lambda i: (i, 0)),
        out_shape=jax.ShapeDtypeStruct((B * M, OC), jnp.float32),
    )(g, samp, w1t, b1r, w2t, b2r, wlp, blr)

    return jnp.transpose(o.reshape(B, M, OC), (0, 2, 1)).reshape(B, OC, hh, ww)
